# baseline (speedup 1.0000x reference)
import jax
import jax.numpy as jnp
from jax.experimental import pallas as pl
from jax.experimental.pallas import tpu as pltpu


def _leaky(x, slope):
    return jnp.where(x > 0, x, slope * x)


def _elu(x):
    return jnp.where(x > 0, x, jnp.exp(x) - 1.0)


def _gat_block_t(x_in, x_feat_dim, W, a_src_row, a_dst_row):
    H = W.shape[1]
    h_t = jax.lax.dot_general(W, x_in, (((0,), (x_feat_dim,)), ((), ())),
                              preferred_element_type=jnp.float32)
    es_row = jnp.dot(a_src_row, h_t, preferred_element_type=jnp.float32)
    ed_row = jnp.dot(a_dst_row, h_t, preferred_element_type=jnp.float32)
    edm = jnp.max(ed_row, axis=1, keepdims=True)

    ed_col = jax.lax.dot_general(h_t, a_dst_row, (((0,), (1,)), ((), ())),
                                 preferred_element_type=jnp.float32)
    Mt = (ed_col > -es_row).astype(jnp.float32)

    p_row = jnp.exp(ed_row - edm)
    q_row = jnp.exp(0.2 * (ed_row - edm))
    ph_t = p_row * h_t
    qh_t = q_row * h_t
    Pt = jnp.concatenate([ph_t, qh_t, p_row, q_row], axis=0)
    At = jax.lax.dot_general(Pt, Mt, (((1,), (0,)), ((), ())),
                             preferred_element_type=jnp.float32)

    tot_qh = jnp.sum(qh_t, axis=1, keepdims=True)
    tot_q = jnp.sum(q_row, axis=1, keepdims=True)

    t_row = es_row + edm
    alpha = jnp.exp(0.8 * jnp.minimum(t_row, 0.0))
    beta = jnp.exp(-0.8 * jnp.maximum(t_row, 0.0))

    pos_h = At[0:H, :]
    neg_h = tot_qh - At[H:2 * H, :]
    pos_1 = At[2 * H:2 * H + 1, :]
    neg_1 = tot_q - At[2 * H + 1:2 * H + 2, :]

    num = alpha * pos_h + beta * neg_h
    den = alpha * pos_1 + beta * neg_1
    return num / den


def _backbone_kernel(x_ref, W1_ref, a1s_ref, a1d_ref, W2_ref, a2s_ref,
                     a2d_ref, out_ref):
    for s in range(x_ref.shape[0]):
        x_nf = x_ref[s]
        h1 = _elu(_gat_block_t(x_nf, 1, W1_ref[...], a1s_ref[...],
                               a1d_ref[...]))
        h2 = _elu(_gat_block_t(h1, 0, W2_ref[...], a2s_ref[...],
                               a2d_ref[...]))
        out_ref[s] = h2.T


def _head_kernel(f_ref, W1_ref, b1_ref, W2_ref, b2_ref, out_ref):
    z = jnp.dot(f_ref[...], W1_ref[...], preferred_element_type=jnp.float32)
    z = _leaky(z + b1_ref[...], 0.01)
    out_ref[...] = jnp.dot(z, W2_ref[...],
                           preferred_element_type=jnp.float32) + b2_ref[...]


@jax.jit
def kernel(x, W1, a1_src, a1_dst, W2, a2_src, a2_dst,
           head_W1, head_b1, head_W2, head_b2):
    B, N, F_in = x.shape
    H1 = W1.shape[1]
    H2 = W2.shape[1]

    a1s = a1_src.reshape(1, H1)
    a1d = a1_dst.reshape(1, H1)
    a2s = a2_src.reshape(1, H2)
    a2d = a2_dst.reshape(1, H2)

    SB = 4
    rep = lambda shape: pl.BlockSpec(shape, lambda b: (0,) * len(shape))
    h2t = pl.pallas_call(
        _backbone_kernel,
        grid=(B // SB,),
        in_specs=[
            pl.BlockSpec((SB, N, F_in), lambda b: (b, 0, 0)),
            rep(W1.shape), rep(a1s.shape), rep(a1d.shape),
            rep(W2.shape), rep(a2s.shape), rep(a2d.shape),
        ],
        out_specs=pl.BlockSpec((SB, N, H2), lambda b: (b, 0, 0)),
        out_shape=jax.ShapeDtypeStruct((B, N, H2), jnp.float32),
        compiler_params=pltpu.CompilerParams(
            dimension_semantics=("arbitrary",)),
    )(x, W1, a1s, a1d, W2, a2s, a2d)

    features = h2t.reshape(B, N * H2)
    pred = pl.pallas_call(
        _head_kernel,
        out_shape=jax.ShapeDtypeStruct((B, 1), jnp.float32),
    )(features, head_W1, head_b1.reshape(1, -1), head_W2,
      head_b2.reshape(1, -1))
    return pred


# speedup vs baseline: 1.4651x; 1.4651x over previous
import jax
import jax.numpy as jnp
from jax.experimental import pallas as pl
from jax.experimental.pallas import tpu as pltpu


def _leaky(x, slope):
    return jnp.where(x > 0, x, slope * x)


def _elu(x):
    return jnp.where(x > 0, x, jnp.exp(x) - 1.0)


def _gat_block_t(x_in, x_feat_dim, W, a_src_row, a_dst_row):
    H = W.shape[1]
    h_t = jax.lax.dot_general(W, x_in, (((0,), (x_feat_dim,)), ((), ())),
                              preferred_element_type=jnp.float32)
    es_row = jnp.dot(a_src_row, h_t, preferred_element_type=jnp.float32)
    ed_row = jnp.dot(a_dst_row, h_t, preferred_element_type=jnp.float32)
    edm = jnp.max(ed_row, axis=1, keepdims=True)

    ed_col = jax.lax.dot_general(h_t, a_dst_row, (((0,), (1,)), ((), ())),
                                 preferred_element_type=jnp.float32)
    Mt = (ed_col > -es_row).astype(jnp.float32)

    p_row = jnp.exp(ed_row - edm)
    q_row = jnp.exp(0.2 * (ed_row - edm))
    ph_t = p_row * h_t
    qh_t = q_row * h_t
    Pt = jnp.concatenate([ph_t, qh_t, p_row, q_row], axis=0)
    At = jax.lax.dot_general(Pt, Mt, (((1,), (0,)), ((), ())),
                             preferred_element_type=jnp.float32)

    tot_qh = jnp.sum(qh_t, axis=1, keepdims=True)
    tot_q = jnp.sum(q_row, axis=1, keepdims=True)

    t_row = es_row + edm
    alpha = jnp.exp(0.8 * jnp.minimum(t_row, 0.0))
    beta = jnp.exp(-0.8 * jnp.maximum(t_row, 0.0))

    pos_h = At[0:H, :]
    neg_h = tot_qh - At[H:2 * H, :]
    pos_1 = At[2 * H:2 * H + 1, :]
    neg_1 = tot_q - At[2 * H + 1:2 * H + 2, :]

    num = alpha * pos_h + beta * neg_h
    den = alpha * pos_1 + beta * neg_1
    return num / den


def _backbone_kernel(xt_ref, W1_ref, a1s_ref, a1d_ref, W2_ref, a2s_ref,
                     a2d_ref, out_ref):
    for s in range(xt_ref.shape[0]):
        xt = xt_ref[s]
        h1 = _elu(_gat_block_t(xt, 0, W1_ref[...], a1s_ref[...],
                               a1d_ref[...]))
        h2 = _elu(_gat_block_t(h1, 0, W2_ref[...], a2s_ref[...],
                               a2d_ref[...]))
        out_ref[:, s, :] = h2


def _head_kernel(h2t_ref, W1_ref, b1_ref, W2_ref, b2_ref, out_ref):
    H2 = h2t_ref.shape[0]
    z = None
    for h in range(H2):
        part = jax.lax.dot_general(
            h2t_ref[h], W1_ref[:, h, :], (((1,), (0,)), ((), ())),
            preferred_element_type=jnp.float32)
        z = part if z is None else z + part
    z = _leaky(z + b1_ref[...], 0.01)
    out_ref[...] = jnp.dot(z, W2_ref[...],
                           preferred_element_type=jnp.float32) + b2_ref[...]


@jax.jit
def kernel(x, W1, a1_src, a1_dst, W2, a2_src, a2_dst,
           head_W1, head_b1, head_W2, head_b2):
    B, N, F_in = x.shape
    H1 = W1.shape[1]
    H2 = W2.shape[1]

    xt = x.transpose(0, 2, 1)
    a1s = a1_src.reshape(1, H1)
    a1d = a1_dst.reshape(1, H1)
    a2s = a2_src.reshape(1, H2)
    a2d = a2_dst.reshape(1, H2)

    SB = 8
    rep = lambda shape: pl.BlockSpec(shape, lambda b: (0,) * len(shape))
    h2t = pl.pallas_call(
        _backbone_kernel,
        grid=(B // SB,),
        in_specs=[
            pl.BlockSpec((SB, F_in, N), lambda b: (b, 0, 0)),
            rep(W1.shape), rep(a1s.shape), rep(a1d.shape),
            rep(W2.shape), rep(a2s.shape), rep(a2d.shape),
        ],
        out_specs=pl.BlockSpec((H2, SB, N), lambda b: (0, b, 0)),
        out_shape=jax.ShapeDtypeStruct((H2, B, N), jnp.float32),
        compiler_params=pltpu.CompilerParams(
            dimension_semantics=("arbitrary",)),
    )(xt, W1, a1s, a1d, W2, a2s, a2d)

    W1v = head_W1.reshape(N, H2, -1)
    pred = pl.pallas_call(
        _head_kernel,
        out_shape=jax.ShapeDtypeStruct((B, 1), jnp.float32),
    )(h2t, W1v, head_b1.reshape(1, -1), head_W2,
      head_b2.reshape(1, -1))
    return pred


# speedup vs baseline: 1.4810x; 1.0109x over previous
import jax
import jax.numpy as jnp
from jax.experimental import pallas as pl
from jax.experimental.pallas import tpu as pltpu


def _leaky(x, slope):
    return jnp.where(x > 0, x, slope * x)


def _elu(x):
    return jnp.where(x > 0, x, jnp.exp(x) - 1.0)


def _gat_block_t(x_in, x_feat_dim, W, a_src_row, a_dst_row):
    H = W.shape[1]
    h_t = jax.lax.dot_general(W, x_in, (((0,), (x_feat_dim,)), ((), ())),
                              preferred_element_type=jnp.float32)
    es_row = jnp.dot(a_src_row, h_t, preferred_element_type=jnp.float32)
    ed_row = jnp.dot(a_dst_row, h_t, preferred_element_type=jnp.float32)
    edm = jnp.max(ed_row, axis=1, keepdims=True)

    ed_col = jax.lax.dot_general(h_t, a_dst_row, (((0,), (1,)), ((), ())),
                                 preferred_element_type=jnp.float32)
    Mt = (ed_col > -es_row).astype(jnp.float32)

    p_row = jnp.exp(ed_row - edm)
    q_row = jnp.exp(0.2 * (ed_row - edm))
    ph_t = p_row * h_t
    qh_t = q_row * h_t
    Pt = jnp.concatenate([ph_t, qh_t, p_row, q_row], axis=0)
    At = jax.lax.dot_general(Pt, Mt, (((1,), (0,)), ((), ())),
                             preferred_element_type=jnp.float32)

    tot_qh = jnp.sum(qh_t, axis=1, keepdims=True)
    tot_q = jnp.sum(q_row, axis=1, keepdims=True)

    t_row = es_row + edm
    alpha = jnp.exp(0.8 * jnp.minimum(t_row, 0.0))
    beta = jnp.exp(-0.8 * jnp.maximum(t_row, 0.0))

    pos_h = At[0:H, :]
    neg_h = tot_qh - At[H:2 * H, :]
    pos_1 = At[2 * H:2 * H + 1, :]
    neg_1 = tot_q - At[2 * H + 1:2 * H + 2, :]

    num = alpha * pos_h + beta * neg_h
    den = alpha * pos_1 + beta * neg_1
    return num / den


def _backbone_kernel(xt_ref, W1_ref, a1s_ref, a1d_ref, W2_ref, a2s_ref,
                     a2d_ref, out_ref):
    for s in range(xt_ref.shape[0]):
        xt = xt_ref[s]
        h1 = _elu(_gat_block_t(xt, 0, W1_ref[...], a1s_ref[...],
                               a1d_ref[...]))
        h2 = _elu(_gat_block_t(h1, 0, W2_ref[...], a2s_ref[...],
                               a2d_ref[...]))
        out_ref[:, s, :] = h2


def _head_kernel(h2t_ref, W1_ref, b1_ref, W2_ref, b2_ref, out_ref):
    H2 = h2t_ref.shape[0]
    z = None
    for h in range(H2):
        part = jax.lax.dot_general(
            h2t_ref[h], W1_ref[:, h, :], (((1,), (0,)), ((), ())),
            preferred_element_type=jnp.float32)
        z = part if z is None else z + part
    z = _leaky(z + b1_ref[...], 0.01)
    out_ref[...] = jnp.dot(z, W2_ref[...],
                           preferred_element_type=jnp.float32) + b2_ref[...]


@jax.jit
def kernel(x, W1, a1_src, a1_dst, W2, a2_src, a2_dst,
           head_W1, head_b1, head_W2, head_b2):
    B, N, F_in = x.shape
    H1 = W1.shape[1]
    H2 = W2.shape[1]

    xt = x.transpose(0, 2, 1)
    a1s = a1_src.reshape(1, H1)
    a1d = a1_dst.reshape(1, H1)
    a2s = a2_src.reshape(1, H2)
    a2d = a2_dst.reshape(1, H2)

    SB = 16
    rep = lambda shape: pl.BlockSpec(shape, lambda b: (0,) * len(shape))
    h2t = pl.pallas_call(
        _backbone_kernel,
        grid=(B // SB,),
        in_specs=[
            pl.BlockSpec((SB, F_in, N), lambda b: (b, 0, 0)),
            rep(W1.shape), rep(a1s.shape), rep(a1d.shape),
            rep(W2.shape), rep(a2s.shape), rep(a2d.shape),
        ],
        out_specs=pl.BlockSpec((H2, SB, N), lambda b: (0, b, 0)),
        out_shape=jax.ShapeDtypeStruct((H2, B, N), jnp.float32),
        compiler_params=pltpu.CompilerParams(
            dimension_semantics=("arbitrary",)),
    )(xt, W1, a1s, a1d, W2, a2s, a2d)

    W1v = head_W1.reshape(N, H2, -1)
    pred = pl.pallas_call(
        _head_kernel,
        out_shape=jax.ShapeDtypeStruct((B, 1), jnp.float32),
    )(h2t, W1v, head_b1.reshape(1, -1), head_W2,
      head_b2.reshape(1, -1))
    return pred


# speedup vs baseline: 1.4819x; 1.0006x over previous
import jax
import jax.numpy as jnp
from jax.experimental import pallas as pl
from jax.experimental.pallas import tpu as pltpu


def _leaky(x, slope):
    return jnp.where(x > 0, x, slope * x)


def _elu(x):
    return jnp.where(x > 0, x, jnp.exp(x) - 1.0)


def _gat_block_t(x_in, x_feat_dim, W, a_src_row, a_dst_row):
    H = W.shape[1]
    h_t = jax.lax.dot_general(W, x_in, (((0,), (x_feat_dim,)), ((), ())),
                              preferred_element_type=jnp.float32)
    es_row = jnp.dot(a_src_row, h_t, preferred_element_type=jnp.float32)
    ed_row = jnp.dot(a_dst_row, h_t, preferred_element_type=jnp.float32)
    edm = jnp.max(ed_row, axis=1, keepdims=True)

    ed_col = jax.lax.dot_general(h_t, a_dst_row, (((0,), (1,)), ((), ())),
                                 preferred_element_type=jnp.float32)
    Mt = (ed_col > -es_row).astype(jnp.float32)

    p_row = jnp.exp(ed_row - edm)
    q_row = jnp.exp(0.2 * (ed_row - edm))
    ph_t = p_row * h_t
    qh_t = q_row * h_t
    Pt = jnp.concatenate([ph_t, qh_t, p_row, q_row], axis=0)
    At = jax.lax.dot_general(Pt, Mt, (((1,), (0,)), ((), ())),
                             preferred_element_type=jnp.float32)

    tot_qh = jnp.sum(qh_t, axis=1, keepdims=True)
    tot_q = jnp.sum(q_row, axis=1, keepdims=True)

    t_row = es_row + edm
    alpha = jnp.exp(0.8 * jnp.minimum(t_row, 0.0))
    beta = jnp.exp(-0.8 * jnp.maximum(t_row, 0.0))

    pos_h = At[0:H, :]
    neg_h = tot_qh - At[H:2 * H, :]
    pos_1 = At[2 * H:2 * H + 1, :]
    neg_1 = tot_q - At[2 * H + 1:2 * H + 2, :]

    num = alpha * pos_h + beta * neg_h
    den = alpha * pos_1 + beta * neg_1
    return num / den


def _backbone_kernel(xt_ref, W1_ref, a1s_ref, a1d_ref, W2_ref, a2s_ref,
                     a2d_ref, out_ref):
    for s in range(xt_ref.shape[0]):
        xt = xt_ref[s]
        h1 = _elu(_gat_block_t(xt, 0, W1_ref[...], a1s_ref[...],
                               a1d_ref[...]))
        h2 = _elu(_gat_block_t(h1, 0, W2_ref[...], a2s_ref[...],
                               a2d_ref[...]))
        out_ref[:, s, :] = h2


def _head_kernel(h2t_ref, W1_ref, b1_ref, W2_ref, b2_ref, out_ref):
    H2 = h2t_ref.shape[0]
    z = None
    for h in range(H2):
        part = jax.lax.dot_general(
            h2t_ref[h], W1_ref[:, h, :], (((1,), (0,)), ((), ())),
            preferred_element_type=jnp.float32)
        z = part if z is None else z + part
    z = _leaky(z + b1_ref[...], 0.01)
    out_ref[...] = jnp.dot(z, W2_ref[...],
                           preferred_element_type=jnp.float32) + b2_ref[...]


@jax.jit
def kernel(x, W1, a1_src, a1_dst, W2, a2_src, a2_dst,
           head_W1, head_b1, head_W2, head_b2):
    B, N, F_in = x.shape
    H1 = W1.shape[1]
    H2 = W2.shape[1]

    xt = x.transpose(0, 2, 1)
    a1s = a1_src.reshape(1, H1)
    a1d = a1_dst.reshape(1, H1)
    a2s = a2_src.reshape(1, H2)
    a2d = a2_dst.reshape(1, H2)

    SB = 16
    rep = lambda shape: pl.BlockSpec(shape, lambda b: (0,) * len(shape))
    h2t = pl.pallas_call(
        _backbone_kernel,
        grid=(B // SB,),
        in_specs=[
            pl.BlockSpec((SB, F_in, N), lambda b: (b, 0, 0)),
            rep(W1.shape), rep(a1s.shape), rep(a1d.shape),
            rep(W2.shape), rep(a2s.shape), rep(a2d.shape),
        ],
        out_specs=pl.BlockSpec((H2, SB, N), lambda b: (0, b, 0)),
        out_shape=jax.ShapeDtypeStruct((H2, B, N), jnp.float32),
        compiler_params=pltpu.CompilerParams(
            dimension_semantics=("parallel",)),
    )(xt, W1, a1s, a1d, W2, a2s, a2d)

    W1v = head_W1.reshape(N, H2, -1)
    pred = pl.pallas_call(
        _head_kernel,
        out_shape=jax.ShapeDtypeStruct((B, 1), jnp.float32),
    )(h2t, W1v, head_b1.reshape(1, -1), head_W2,
      head_b2.reshape(1, -1))
    return pred
